# Initial kernel scaffold; baseline (speedup 1.0000x reference)
import jax
import jax.numpy as jnp
from jax.experimental import pallas as pl


def kernel(x, token_table, type_table, ln_gamma, ln_beta):
    raise NotImplementedError("write your pallas kernel here")


# speedup vs baseline: 2.3042x; 2.3042x over previous
import functools
import math

import jax
import jax.numpy as jnp
from jax import lax
from jax.experimental import pallas as pl
from jax.experimental.pallas import tpu as pltpu
from jax.experimental.pallas import tpu_sc as plsc

NC = 2
NS = 16
NW = NC * NS
N_TOK = 4096 * 200
PER_W = N_TOK // NW
CHUNK = 256
NCH = PER_W // CHUNK
IDX_ROWS = PER_W // 128
D_TOK = 56
D_MODEL = 64


def _rsqrt(y):
    i = lax.bitcast_convert_type(y, jnp.int32)
    i = jnp.int32(0x5F3759DF) - lax.shift_right_arithmetic(i, 1)
    r = lax.bitcast_convert_type(i, jnp.float32)
    for _ in range(3):
        r = r * (1.5 - 0.5 * y * r * r)
    return r


def _body(x_r, tab_r, typ_r, gm_r, bt_r, out_r,
          idxv, rowsb, outb, typev, gbv, bbv, sem_g):
    wid = lax.axis_index("s") * NC + lax.axis_index("c")
    pltpu.sync_copy(x_r.at[wid], idxv)
    pltpu.sync_copy(typ_r, typev)
    pltpu.sync_copy(gm_r, gbv)
    pltpu.sync_copy(bt_r, bbv)
    wbase = wid * PER_W
    iota = lax.iota(jnp.int32, 16)

    def group(g, k):
        xv = idxv[2 * k + g // 8, pl.ds((g % 8) * 16, 16)]
        tix = ((xv >= 50000).astype(jnp.int32)
               + (xv >= 60000).astype(jnp.int32)
               + (xv >= 80000).astype(jnp.int32))
        rowvec = g * 16 + iota
        s = jnp.zeros((16,), jnp.float32)
        q = jnp.zeros((16,), jnp.float32)
        for d in range(D_TOK):
            v = plsc.load_gather(rowsb, [rowvec, jnp.full((16,), d, jnp.int32)])
            s = s + v
            q = q + v * v
        tcol = tix * 8
        tvs = []
        for c in range(8):
            v = plsc.load_gather(typev, [tcol + c])
            tvs.append(v)
            s = s + v
            q = q + v * v
        mean = s * (1.0 / D_MODEL)
        var = q * (1.0 / D_MODEL) - mean * mean
        rq = _rsqrt(var + 1e-5)
        for d in range(D_TOK):
            v = plsc.load_gather(rowsb, [rowvec, jnp.full((16,), d, jnp.int32)])
            o = (v - mean) * rq
            o = o * gbv[d] + bbv[d]
            plsc.store_scatter(outb, [rowvec, jnp.full((16,), d, jnp.int32)], o)
        for c in range(8):
            o = (tvs[c] - mean) * rq
            o = o * gbv[D_TOK + c] + bbv[D_TOK + c]
            plsc.store_scatter(
                outb, [rowvec, jnp.full((16,), D_TOK + c, jnp.int32)], o)
        return k

    def chunk(k, _):
        cp0 = pltpu.async_copy(tab_r.at[idxv.at[2 * k]],
                               rowsb.at[pl.ds(0, 128)], sem_g)
        cp1 = pltpu.async_copy(tab_r.at[idxv.at[2 * k + 1]],
                               rowsb.at[pl.ds(128, 128)], sem_g)
        cp0.wait()
        cp1.wait()
        lax.fori_loop(0, CHUNK // 16, group, k)
        pltpu.sync_copy(outb, out_r.at[pl.ds(wbase + k * CHUNK, CHUNK)])
        return _

    lax.fori_loop(0, NCH, chunk, 0)


_mesh = plsc.VectorSubcoreMesh(core_axis_name="c", subcore_axis_name="s",
                               num_cores=NC, num_subcores=NS)

_sc_call = functools.partial(
    pl.kernel,
    out_type=jax.ShapeDtypeStruct((N_TOK, D_MODEL), jnp.float32),
    mesh=_mesh,
    compiler_params=pltpu.CompilerParams(
        needs_layout_passes=False, use_tc_tiling_on_sc=False),
    scratch_types=[
        pltpu.VMEM((IDX_ROWS, 128), jnp.int32),
        pltpu.VMEM((CHUNK, D_TOK), jnp.float32),
        pltpu.VMEM((CHUNK, D_MODEL), jnp.float32),
        pltpu.VMEM((32,), jnp.float32),
        pltpu.VMEM((D_MODEL, 16), jnp.float32),
        pltpu.VMEM((D_MODEL, 16), jnp.float32),
        pltpu.SemaphoreType.DMA,
    ],
)(_body)


def kernel(x, token_table, type_table, ln_gamma, ln_beta):
    scale = math.sqrt(D_MODEL)
    xr = x.reshape(NW, IDX_ROWS, 128)
    g_b = jnp.broadcast_to((ln_gamma * scale)[:, None], (D_MODEL, 16))
    b_b = jnp.broadcast_to((ln_beta * scale)[:, None], (D_MODEL, 16))
    out = _sc_call(xr, token_table, type_table.reshape(32), g_b, b_b)
    return out.reshape(4096, 200, D_MODEL)


# speedup vs baseline: 2.4815x; 1.0769x over previous
import functools
import math

import jax
import jax.numpy as jnp
from jax import lax
from jax.experimental import pallas as pl
from jax.experimental.pallas import tpu as pltpu
from jax.experimental.pallas import tpu_sc as plsc

NC = 2
NS = 16
NW = NC * NS
N_TOK = 4096 * 200
PER_W = N_TOK // NW
CHUNK = 256
NCH = PER_W // CHUNK
IDX_ROWS = PER_W // 128
D_TOK = 56
D_MODEL = 64


def _rsqrt(y):
    i = lax.bitcast_convert_type(y, jnp.int32)
    i = jnp.int32(0x5F3759DF) - lax.shift_right_arithmetic(i, 1)
    r = lax.bitcast_convert_type(i, jnp.float32)
    for _ in range(3):
        r = r * (1.5 - 0.5 * y * r * r)
    return r


def _body(x_r, tab_r, typ_r, gm_r, bt_r, out_r,
          idxv, rowsb0, rowsb1, outb0, outb1, typev, gbv, bbv,
          sem_g0, sem_g1, sem_o0, sem_o1):
    wid = lax.axis_index("s") * NC + lax.axis_index("c")
    pltpu.sync_copy(x_r.at[wid], idxv)
    pltpu.sync_copy(typ_r, typev)
    pltpu.sync_copy(gm_r, gbv)
    pltpu.sync_copy(bt_r, bbv)
    wbase = wid * PER_W
    iota = lax.iota(jnp.int32, 16)

    def group(g, k, rb=None, ob=None):
        xv = idxv[2 * k + g // 8, pl.ds((g % 8) * 16, 16)]
        tix = ((xv >= 50000).astype(jnp.int32)
               + (xv >= 60000).astype(jnp.int32)
               + (xv >= 80000).astype(jnp.int32))
        rowvec = g * 16 + iota
        s = jnp.zeros((16,), jnp.float32)
        q = jnp.zeros((16,), jnp.float32)
        for d in range(D_TOK):
            v = plsc.load_gather(rb, [rowvec, jnp.full((16,), d, jnp.int32)])
            s = s + v
            q = q + v * v
        tcol = tix * 8
        tvs = []
        for c in range(8):
            v = plsc.load_gather(typev, [tcol + c])
            tvs.append(v)
            s = s + v
            q = q + v * v
        mean = s * (1.0 / D_MODEL)
        var = q * (1.0 / D_MODEL) - mean * mean
        rq = _rsqrt(var + 1e-5)
        for d in range(D_TOK):
            v = plsc.load_gather(rb, [rowvec, jnp.full((16,), d, jnp.int32)])
            o = (v - mean) * rq
            o = o * gbv[d] + bbv[d]
            plsc.store_scatter(ob, [rowvec, jnp.full((16,), d, jnp.int32)], o)
        for c in range(8):
            o = (tvs[c] - mean) * rq
            o = o * gbv[D_TOK + c] + bbv[D_TOK + c]
            plsc.store_scatter(
                ob, [rowvec, jnp.full((16,), D_TOK + c, jnp.int32)], o)
        return k

    rows = (rowsb0, rowsb1)
    outs = (outb0, outb1)
    sgs = (sem_g0, sem_g1)
    sos = (sem_o0, sem_o1)

    def issue_gather(k, b):
        pltpu.async_copy(tab_r.at[idxv.at[2 * k]],
                         rows[b].at[pl.ds(0, 128)], sgs[b])
        pltpu.async_copy(tab_r.at[idxv.at[2 * k + 1]],
                         rows[b].at[pl.ds(128, 128)], sgs[b])

    def wait_gather(b):
        for h in range(2):
            pltpu.make_async_copy(tab_r.at[pl.ds(0, 128)],
                                  rows[b].at[pl.ds(h * 128, 128)],
                                  sgs[b]).wait()

    def issue_out(k, b):
        pltpu.async_copy(outs[b], out_r.at[pl.ds(wbase + k * CHUNK, CHUNK)],
                         sos[b])

    def wait_out(b):
        pltpu.make_async_copy(outs[b], out_r.at[pl.ds(wbase, CHUNK)],
                              sos[b]).wait()

    def compute(k, b):
        lax.fori_loop(0, CHUNK // 16, functools.partial(group, rb=rows[b],
                                                        ob=outs[b]), k)

    issue_gather(0, 0)
    issue_gather(1, 1)
    for k in (0, 1):
        b = k & 1
        wait_gather(b)
        compute(k, b)
        issue_out(k, b)
        issue_gather(k + 2, b)

    def middle(j, _):
        for b in range(2):
            k = 2 * j + b
            wait_gather(b)
            wait_out(b)
            compute(k, b)
            issue_out(k, b)
            issue_gather(k + 2, b)
        return _

    lax.fori_loop(1, NCH // 2 - 1, middle, 0)

    for k in (NCH - 2, NCH - 1):
        b = k & 1
        wait_gather(b)
        wait_out(b)
        compute(k, b)
        issue_out(k, b)
    wait_out(0)
    wait_out(1)


_mesh = plsc.VectorSubcoreMesh(core_axis_name="c", subcore_axis_name="s",
                               num_cores=NC, num_subcores=NS)

_sc_call = functools.partial(
    pl.kernel,
    out_type=jax.ShapeDtypeStruct((N_TOK, D_MODEL), jnp.float32),
    mesh=_mesh,
    compiler_params=pltpu.CompilerParams(
        needs_layout_passes=False, use_tc_tiling_on_sc=False),
    scratch_types=[
        pltpu.VMEM((IDX_ROWS, 128), jnp.int32),
        pltpu.VMEM((CHUNK, D_TOK), jnp.float32),
        pltpu.VMEM((CHUNK, D_TOK), jnp.float32),
        pltpu.VMEM((CHUNK, D_MODEL), jnp.float32),
        pltpu.VMEM((CHUNK, D_MODEL), jnp.float32),
        pltpu.VMEM((32,), jnp.float32),
        pltpu.VMEM((D_MODEL, 16), jnp.float32),
        pltpu.VMEM((D_MODEL, 16), jnp.float32),
        pltpu.SemaphoreType.DMA,
        pltpu.SemaphoreType.DMA,
        pltpu.SemaphoreType.DMA,
        pltpu.SemaphoreType.DMA,
    ],
)(_body)


def kernel(x, token_table, type_table, ln_gamma, ln_beta):
    scale = math.sqrt(D_MODEL)
    xr = x.reshape(NW, IDX_ROWS, 128)
    g_b = jnp.broadcast_to((ln_gamma * scale)[:, None], (D_MODEL, 16))
    b_b = jnp.broadcast_to((ln_beta * scale)[:, None], (D_MODEL, 16))
    out = _sc_call(xr, token_table, type_table.reshape(32), g_b, b_b)
    return out.reshape(4096, 200, D_MODEL)


# speedup vs baseline: 5.5718x; 2.2453x over previous
import functools
import math

import jax
import jax.numpy as jnp
from jax import lax
from jax.experimental import pallas as pl
from jax.experimental.pallas import tpu as pltpu
from jax.experimental.pallas import tpu_sc as plsc

NC = 2
NS = 16
NW = NC * NS
N_TOK = 4096 * 200
PER_W = N_TOK // NW
CHUNK = 256
NCH = PER_W // CHUNK
IDX_ROWS = PER_W // 128
D_MODEL = 64
UNROLL = 4


def _rsqrt(y):
    i = lax.bitcast_convert_type(y, jnp.int32)
    i = jnp.int32(0x5F3759DF) - lax.shift_right_arithmetic(i, 1)
    r = lax.bitcast_convert_type(i, jnp.float32)
    for _ in range(2):
        r = r * (1.5 - 0.5 * y * r * r)
    return r


def _body(x_r, tab_r, gm_r, bt_r, out_r,
          idxv, rows0, rows1, outb0, outb1, gbv, bbv,
          sem_g0, sem_g1, sem_o0, sem_o1):
    wid = lax.axis_index("s") * NC + lax.axis_index("c")
    pltpu.sync_copy(x_r.at[wid], idxv)
    pltpu.sync_copy(gm_r, gbv)
    pltpu.sync_copy(bt_r, bbv)
    wbase = wid * PER_W
    gs = [gbv[j] for j in range(4)]
    bs = [bbv[j] for j in range(4)]

    rows = (rows0, rows1)
    outs = (outb0, outb1)
    sgs = (sem_g0, sem_g1)
    sos = (sem_o0, sem_o1)

    def issue_gather(k, b):
        pltpu.async_copy(tab_r.at[idxv.at[2 * k]],
                         rows[b].at[pl.ds(0, 128)], sgs[b])
        pltpu.async_copy(tab_r.at[idxv.at[2 * k + 1]],
                         rows[b].at[pl.ds(128, 128)], sgs[b])

    def wait_gather(b):
        for h in range(2):
            pltpu.make_async_copy(tab_r.at[pl.ds(0, 128)],
                                  rows[b].at[pl.ds(h * 128, 128)],
                                  sgs[b]).wait()

    def issue_out(k, b):
        pltpu.async_copy(outs[b], out_r.at[pl.ds(wbase + k * CHUNK, CHUNK)],
                         sos[b])

    def wait_out(b):
        pltpu.make_async_copy(outs[b], out_r.at[pl.ds(wbase, CHUNK)],
                              sos[b]).wait()

    def tokens(ti, k, rb=None, ob=None):
        for u in range(UNROLL):
            t = UNROLL * ti + u
            vs = [rb[t, pl.ds(j * 16, 16)] for j in range(4)]
            s4 = (vs[0] + vs[1]) + (vs[2] + vs[3])
            q4 = (vs[0] * vs[0] + vs[1] * vs[1]) + (vs[2] * vs[2]
                                                    + vs[3] * vs[3])
            mean = lax.broadcast_in_dim(jnp.sum(s4), (16,), ()) * (1.0 / 64)
            qv = lax.broadcast_in_dim(jnp.sum(q4), (16,), ()) * (1.0 / 64)
            rq = _rsqrt(qv - mean * mean + 1e-5)
            for j in range(4):
                o = (vs[j] - mean) * (gs[j] * rq) + bs[j]
                ob[t, pl.ds(j * 16, 16)] = o
        return k

    def compute(k, b):
        lax.fori_loop(0, CHUNK // UNROLL,
                      functools.partial(tokens, rb=rows[b], ob=outs[b]), k)

    issue_gather(0, 0)
    issue_gather(1, 1)
    for k in (0, 1):
        b = k & 1
        wait_gather(b)
        compute(k, b)
        issue_out(k, b)
        issue_gather(k + 2, b)

    def middle(j, _):
        for b in range(2):
            k = 2 * j + b
            wait_gather(b)
            wait_out(b)
            compute(k, b)
            issue_out(k, b)
            issue_gather(k + 2, b)
        return _

    lax.fori_loop(1, NCH // 2 - 1, middle, 0)

    for k in (NCH - 2, NCH - 1):
        b = k & 1
        wait_gather(b)
        wait_out(b)
        compute(k, b)
        issue_out(k, b)
    wait_out(0)
    wait_out(1)


_mesh = plsc.VectorSubcoreMesh(core_axis_name="c", subcore_axis_name="s",
                               num_cores=NC, num_subcores=NS)

_sc_call = functools.partial(
    pl.kernel,
    out_type=jax.ShapeDtypeStruct((N_TOK, D_MODEL), jnp.float32),
    mesh=_mesh,
    compiler_params=pltpu.CompilerParams(
        needs_layout_passes=False, use_tc_tiling_on_sc=False),
    scratch_types=[
        pltpu.VMEM((IDX_ROWS, 128), jnp.int32),
        pltpu.VMEM((CHUNK, D_MODEL), jnp.float32),
        pltpu.VMEM((CHUNK, D_MODEL), jnp.float32),
        pltpu.VMEM((CHUNK, D_MODEL), jnp.float32),
        pltpu.VMEM((CHUNK, D_MODEL), jnp.float32),
        pltpu.VMEM((4, 16), jnp.float32),
        pltpu.VMEM((4, 16), jnp.float32),
        pltpu.SemaphoreType.DMA,
        pltpu.SemaphoreType.DMA,
        pltpu.SemaphoreType.DMA,
        pltpu.SemaphoreType.DMA,
    ],
)(_body)


def kernel(x, token_table, type_table, ln_gamma, ln_beta):
    scale = math.sqrt(D_MODEL)
    xr = x.reshape(NW, IDX_ROWS, 128)
    ids = lax.iota(jnp.int32, token_table.shape[0])
    tix = ((ids >= 50000).astype(jnp.int32)
           + (ids >= 60000).astype(jnp.int32)
           + (ids >= 80000).astype(jnp.int32))
    table64 = jnp.concatenate([token_table, jnp.take(type_table, tix, 0)],
                              axis=1)
    g_b = (ln_gamma * scale).reshape(4, 16)
    b_b = (ln_beta * scale).reshape(4, 16)
    out = _sc_call(xr, table64, g_b, b_b)
    return out.reshape(4096, 200, D_MODEL)


# speedup vs baseline: 9.3345x; 1.6753x over previous
import functools
import math

import jax
import jax.numpy as jnp
from jax import lax
from jax.experimental import pallas as pl
from jax.experimental.pallas import tpu as pltpu
from jax.experimental.pallas import tpu_sc as plsc

NC = 2
NS = 16
NW = NC * NS
N_TOK = 4096 * 200
PER_W = N_TOK // NW
CHUNK = 128
NCH = PER_W // CHUNK
IDX_ROWS = PER_W // 128
D_MODEL = 64
UNROLL = 4


def _rsqrt(y):
    i = lax.bitcast_convert_type(y, jnp.int32)
    i = jnp.int32(0x5F3759DF) - lax.shift_right_arithmetic(i, 1)
    r = lax.bitcast_convert_type(i, jnp.float32)
    for _ in range(2):
        r = r * (1.5 - 0.5 * y * r * r)
    return r


def _body(x_r, tab_r, gb_r, out_r,
          idxv, rows0, rows1, outb0, outb1, gbv,
          sem_g0, sem_g1, sem_o0, sem_o1):
    wid = lax.axis_index("s") * NC + lax.axis_index("c")
    pltpu.sync_copy(x_r.at[wid], idxv)
    pltpu.sync_copy(gb_r, gbv)
    wbase = wid * PER_W
    gs = [gbv[0, pl.ds(j * 16, 16)] for j in range(4)]
    bs = [gbv[0, pl.ds(64 + j * 16, 16)] for j in range(4)]

    rows = (rows0, rows1)
    outs = (outb0, outb1)
    sgs = (sem_g0, sem_g1)
    sos = (sem_o0, sem_o1)

    def issue_gather(k, b):
        pltpu.async_copy(tab_r.at[idxv.at[k]], rows[b], sgs[b])

    def wait_gather(b):
        pltpu.make_async_copy(tab_r.at[pl.ds(0, 128)], rows[b],
                              sgs[b]).wait()

    def issue_out(k, b):
        pltpu.async_copy(outs[b], out_r.at[pl.ds(wbase + k * CHUNK, CHUNK)],
                         sos[b])

    def wait_out(b):
        pltpu.make_async_copy(outs[b], out_r.at[pl.ds(wbase, CHUNK)],
                              sos[b]).wait()

    def tokens(ti, k, rb=None, ob=None):
        for u in range(UNROLL):
            t = UNROLL * ti + u
            vs = [rb[t, pl.ds(j * 16, 16)] for j in range(4)]
            s4 = (vs[0] + vs[1]) + (vs[2] + vs[3])
            q4 = (vs[0] * vs[0] + vs[1] * vs[1]) + (vs[2] * vs[2]
                                                    + vs[3] * vs[3])
            mean = lax.broadcast_in_dim(jnp.sum(s4), (16,), ()) * (1.0 / 64)
            qv = lax.broadcast_in_dim(jnp.sum(q4), (16,), ()) * (1.0 / 64)
            rq = _rsqrt(qv - mean * mean + 1e-5)
            for j in range(4):
                o = (vs[j] - mean) * (gs[j] * rq) + bs[j]
                ob[t, pl.ds(j * 16, 16)] = o
        return k

    def compute(k, b):
        lax.fori_loop(0, CHUNK // UNROLL,
                      functools.partial(tokens, rb=rows[b], ob=outs[b]), k)

    issue_gather(0, 0)
    issue_gather(1, 1)
    for k in (0, 1):
        b = k & 1
        wait_gather(b)
        compute(k, b)
        issue_out(k, b)
        issue_gather(k + 2, b)

    def middle(j, _):
        for b in range(2):
            k = 2 * j + b
            wait_gather(b)
            wait_out(b)
            compute(k, b)
            issue_out(k, b)
            issue_gather(k + 2, b)
        return _

    lax.fori_loop(1, NCH // 2 - 1, middle, 0)

    for k in (NCH - 2, NCH - 1):
        b = k & 1
        wait_gather(b)
        wait_out(b)
        compute(k, b)
        issue_out(k, b)
    wait_out(0)
    wait_out(1)


_mesh = plsc.VectorSubcoreMesh(core_axis_name="c", subcore_axis_name="s",
                               num_cores=NC, num_subcores=NS)

_sc_call = functools.partial(
    pl.kernel,
    out_type=jax.ShapeDtypeStruct((N_TOK, D_MODEL), jnp.float32),
    mesh=_mesh,
    compiler_params=pltpu.CompilerParams(
        needs_layout_passes=False, use_tc_tiling_on_sc=True),
    scratch_types=[
        pltpu.VMEM((IDX_ROWS, 128), jnp.int32),
        pltpu.VMEM((CHUNK, 128), jnp.float32),
        pltpu.VMEM((CHUNK, 128), jnp.float32),
        pltpu.VMEM((CHUNK, D_MODEL), jnp.float32),
        pltpu.VMEM((CHUNK, D_MODEL), jnp.float32),
        pltpu.VMEM((1, 128), jnp.float32),
        pltpu.SemaphoreType.DMA,
        pltpu.SemaphoreType.DMA,
        pltpu.SemaphoreType.DMA,
        pltpu.SemaphoreType.DMA,
    ],
)(_body)


def kernel(x, token_table, type_table, ln_gamma, ln_beta):
    scale = math.sqrt(D_MODEL)
    xr = x.reshape(NW, IDX_ROWS, 128)
    nrows = token_table.shape[0]
    ids = lax.iota(jnp.int32, nrows)[:, None]
    type_rows = jnp.zeros((nrows, 8), jnp.float32)
    for i, lo in enumerate((0, 50000, 60000, 80000)):
        type_rows = jnp.where(
            (ids >= lo), jnp.broadcast_to(type_table[i], (nrows, 8)),
            type_rows)
    table128 = jnp.concatenate(
        [token_table, type_rows, jnp.zeros((nrows, 64), jnp.float32)], axis=1)
    g_b = jnp.concatenate([ln_gamma, ln_beta]).reshape(1, 128) * scale
    out = _sc_call(xr, table128, g_b)
    return out.reshape(4096, 200, D_MODEL)
